# baseline (speedup 1.0000x reference)
import functools

import jax
import jax.numpy as jnp
from jax.experimental import pallas as pl
from jax.experimental.pallas import tpu as pltpu

B, N, G, K = 8, 16384, 256, 32
C_ENC = 384
MASK_RATIO = 0.6
NUM_MASK = int(MASK_RATIO * G)
G_VIS = G - NUM_MASK


def _fps_body(xt_ref, idx_ref, cents_ref, dists_ref):
    x = xt_ref[0]
    y = xt_ref[1]
    z = xt_ref[2]
    lane = jax.lax.broadcasted_iota(jnp.int32, (B, N), 1)
    glane = jax.lax.broadcasted_iota(jnp.int32, (B, G), 1)
    dists_ref[...] = jnp.full((B, N), 1e10, jnp.float32)
    idx_ref[...] = jnp.zeros((B, G), jnp.int32)
    cents_ref[...] = jnp.zeros((3, B, G), jnp.float32)

    def body(i, carry):
        far_i, cx, cy, cz = carry
        at_i = (glane == i).astype(jnp.int32)
        at_f = at_i.astype(jnp.float32)
        idx_ref[...] = idx_ref[...] + at_i * far_i
        cents_ref[0] = cents_ref[0] + at_f * cx
        cents_ref[1] = cents_ref[1] + at_f * cy
        cents_ref[2] = cents_ref[2] + at_f * cz
        dx = x - cx
        dy = y - cy
        dz = z - cz
        d = dx * dx + dy * dy + dz * dz
        dists = jnp.minimum(dists_ref[...], d)
        dists_ref[...] = dists
        m = jnp.max(dists, axis=1, keepdims=True)
        elig = dists == m
        nfar = jnp.min(jnp.where(elig, lane, N), axis=1, keepdims=True)
        sel = lane == nfar
        ncx = jnp.sum(jnp.where(sel, x, 0.0), axis=1, keepdims=True)
        ncy = jnp.sum(jnp.where(sel, y, 0.0), axis=1, keepdims=True)
        ncz = jnp.sum(jnp.where(sel, z, 0.0), axis=1, keepdims=True)
        return nfar, ncx, ncy, ncz

    sel0 = lane == 0
    cx0 = jnp.sum(jnp.where(sel0, x, 0.0), axis=1, keepdims=True)
    cy0 = jnp.sum(jnp.where(sel0, y, 0.0), axis=1, keepdims=True)
    cz0 = jnp.sum(jnp.where(sel0, z, 0.0), axis=1, keepdims=True)
    far0 = jnp.zeros((B, 1), jnp.int32)
    jax.lax.fori_loop(0, G, body, (far0, cx0, cy0, cz0))


def _fps_centers(xyz):
    xt = jnp.transpose(xyz, (2, 0, 1))
    idx, cents = pl.pallas_call(
        _fps_body,
        in_specs=[pl.BlockSpec((3, B, N), lambda: (0, 0, 0))],
        out_specs=[
            pl.BlockSpec((B, G), lambda: (0, 0)),
            pl.BlockSpec((3, B, G), lambda: (0, 0, 0)),
        ],
        out_shape=[
            jax.ShapeDtypeStruct((B, G), jnp.int32),
            jax.ShapeDtypeStruct((3, B, G), jnp.float32),
        ],
        scratch_shapes=[pltpu.VMEM((B, N), jnp.float32)],
    )(xt)
    center = jnp.transpose(cents, (1, 2, 0))
    return idx, center


GT = 64


def _dist_body(xt_ref, c_ref, d_ref):
    x = xt_ref[0, 0]
    y = xt_ref[1, 0]
    z = xt_ref[2, 0]
    c = c_ref[0]
    dx = c[:, 0:1] - x
    dy = c[:, 1:2] - y
    dz = c[:, 2:3] - z
    d_ref[0] = (dx * dx + dy * dy) + dz * dz


def _distances(xt, center):
    xt = xt.reshape(3, B, 1, N)
    return pl.pallas_call(
        _dist_body,
        grid=(B, G // GT),
        in_specs=[
            pl.BlockSpec((3, 1, 1, N), lambda b, g: (0, b, 0, 0)),
            pl.BlockSpec((1, GT, 3), lambda b, g: (b, g, 0)),
        ],
        out_specs=pl.BlockSpec((1, GT, N), lambda b, g: (b, g, 0)),
        out_shape=jax.ShapeDtypeStruct((B, G, N), jnp.float32),
    )(xt, center)


RW = 8
BIG = 1e30


def _extract_body(vals_ref, out_ref, sc_ref):
    W = vals_ref.shape[1]
    sc_ref[...] = vals_ref[...]
    lane = jax.lax.broadcasted_iota(jnp.int32, (RW, W), 1)
    olane = jax.lax.broadcasted_iota(jnp.int32, (RW, 128), 1)
    out_ref[...] = jnp.zeros((RW, 128), jnp.int32)

    def body(j, _):
        vals = sc_ref[...]
        m = jnp.min(vals, axis=1, keepdims=True)
        elig = vals == m
        g = jnp.min(jnp.where(elig, lane, N), axis=1, keepdims=True)
        out_ref[...] = out_ref[...] + (olane == j).astype(jnp.int32) * g
        sc_ref[...] = jnp.where(lane == g, BIG, vals)
        return 0

    jax.lax.fori_loop(0, K, body, 0)


def _topk_full(d):
    R, W = d.shape
    out = pl.pallas_call(
        _extract_body,
        grid=(R // RW,),
        in_specs=[pl.BlockSpec((RW, W), lambda i: (i, 0))],
        out_specs=pl.BlockSpec((RW, 128), lambda i: (i, 0)),
        out_shape=jax.ShapeDtypeStruct((R, 128), jnp.int32),
        scratch_shapes=[pltpu.VMEM((RW, W), jnp.float32)],
    )(d)
    return out[:, :K]


def _batchnorm(x, gamma, beta):
    mu = jnp.mean(x, axis=(0, 1), keepdims=True)
    var = jnp.var(x, axis=(0, 1), keepdims=True)
    return (x - mu) / jnp.sqrt(var + 1e-5) * gamma + beta


def _embedder(pg, W1, b1, g1, be1, W2, b2, W3, b3, g2, be2, W4, b4):
    bg, n, _ = pg.shape
    f = pg @ W1.T + b1
    f = jax.nn.relu(_batchnorm(f, g1, be1))
    f = f @ W2.T + b2
    fg = jnp.max(f, axis=1, keepdims=True)
    f = jnp.concatenate([jnp.broadcast_to(fg, (bg, n, fg.shape[-1])), f], axis=-1)
    f = f @ W3.T + b3
    f = jax.nn.relu(_batchnorm(f, g2, be2))
    f = f @ W4.T + b4
    return jnp.max(f, axis=1)


def _make_mask():
    base = jnp.concatenate([jnp.zeros(G - NUM_MASK), jnp.ones(NUM_MASK)])
    keys = jax.random.split(jax.random.key(42), B)
    mask = jax.vmap(lambda k: jax.random.permutation(k, base))(keys)
    return mask > 0.5


def _pos_mlp_kernel(mc_ref, pw1_ref, pb1_ref, pw2_ref, pb2_ref, out_ref):
    mc = mc_ref[0]
    hh = jnp.dot(mc, pw1_ref[...], preferred_element_type=jnp.float32) + pb1_ref[...]
    g = hh * 0.5 * (1.0 + jax.lax.erf(hh * jnp.float32(0.7071067811865476)))
    out_ref[0] = jnp.dot(g, pw2_ref[...], preferred_element_type=jnp.float32) + pb2_ref[...]


def _pos_mlp(mc, PW1, pb1, PW2, pb2):
    GP = 128
    mcp = jnp.zeros((B, GP, 8), jnp.float32).at[:, :G_VIS, :3].set(mc)
    w1 = jnp.zeros((8, 128), jnp.float32).at[:3, :].set(PW1.T)
    out = pl.pallas_call(
        _pos_mlp_kernel,
        grid=(B,),
        in_specs=[
            pl.BlockSpec((1, GP, 8), lambda i: (i, 0, 0)),
            pl.BlockSpec((8, 128), lambda i: (0, 0)),
            pl.BlockSpec((128,), lambda i: (0,)),
            pl.BlockSpec((128, C_ENC), lambda i: (0, 0)),
            pl.BlockSpec((C_ENC,), lambda i: (0,)),
        ],
        out_specs=pl.BlockSpec((1, GP, C_ENC), lambda i: (i, 0, 0)),
        out_shape=jax.ShapeDtypeStruct((B, GP, C_ENC), jnp.float32),
    )(mcp, w1, pb1, PW2.T, pb2)
    return out[:, :G_VIS, :]


def kernel(xyz, W1, b1, g1, be1, W2, b2, W3, b3, g2, be2, W4, b4, PW1, pb1, PW2, pb2):
    xt = jnp.transpose(xyz, (2, 0, 1))
    _, center = _fps_centers(xyz)
    d = _distances(xt, center)
    knn_idx = _topk_full(d.reshape(B * G, N)).reshape(B, G, K)
    neighborhood = jax.vmap(lambda pts, ind: pts[ind])(xyz, knn_idx)
    neighborhood = neighborhood - center[:, :, None, :]
    mask = _make_mask()
    vis_idx = jnp.argsort(mask.astype(jnp.int32), axis=1)[:, :G_VIS]
    tok_all = _embedder(neighborhood.reshape(B * G, K, 3), W1, b1, g1, be1, W2, b2, W3, b3, g2, be2, W4, b4).reshape(B, G, C_ENC)
    tokens = jnp.take_along_axis(tok_all, vis_idx[:, :, None], axis=1)
    mc = jnp.take_along_axis(center, vis_idx[:, :, None], axis=1)
    pos = _pos_mlp(mc, PW1, pb1, PW2, pb2)
    return tokens, pos, mask, center, neighborhood


# speedup vs baseline: 1.1095x; 1.1095x over previous
import functools

import jax
import jax.numpy as jnp
from jax.experimental import pallas as pl
from jax.experimental.pallas import tpu as pltpu
from jax.experimental.pallas import tpu_sc as plsc

B, N, G, K = 8, 16384, 256, 32
C_ENC = 384
MASK_RATIO = 0.6
NUM_MASK = int(MASK_RATIO * G)
G_VIS = G - NUM_MASK


def _fps_body(xt_ref, idx_ref, cents_ref, dists_ref):
    x = xt_ref[0]
    y = xt_ref[1]
    z = xt_ref[2]
    lane = jax.lax.broadcasted_iota(jnp.int32, (B, N), 1)
    glane = jax.lax.broadcasted_iota(jnp.int32, (B, G), 1)
    dists_ref[...] = jnp.full((B, N), 1e10, jnp.float32)
    idx_ref[...] = jnp.zeros((B, G), jnp.int32)
    cents_ref[...] = jnp.zeros((3, B, G), jnp.float32)

    def body(i, carry):
        far_i, cx, cy, cz = carry
        at_i = (glane == i).astype(jnp.int32)
        at_f = at_i.astype(jnp.float32)
        idx_ref[...] = idx_ref[...] + at_i * far_i
        cents_ref[0] = cents_ref[0] + at_f * cx
        cents_ref[1] = cents_ref[1] + at_f * cy
        cents_ref[2] = cents_ref[2] + at_f * cz
        dx = x - cx
        dy = y - cy
        dz = z - cz
        d = dx * dx + dy * dy + dz * dz
        dists = jnp.minimum(dists_ref[...], d)
        dists_ref[...] = dists
        m = jnp.max(dists, axis=1, keepdims=True)
        elig = dists == m
        nfar = jnp.min(jnp.where(elig, lane, N), axis=1, keepdims=True)
        sel = lane == nfar
        ncx = jnp.sum(jnp.where(sel, x, 0.0), axis=1, keepdims=True)
        ncy = jnp.sum(jnp.where(sel, y, 0.0), axis=1, keepdims=True)
        ncz = jnp.sum(jnp.where(sel, z, 0.0), axis=1, keepdims=True)
        return nfar, ncx, ncy, ncz

    sel0 = lane == 0
    cx0 = jnp.sum(jnp.where(sel0, x, 0.0), axis=1, keepdims=True)
    cy0 = jnp.sum(jnp.where(sel0, y, 0.0), axis=1, keepdims=True)
    cz0 = jnp.sum(jnp.where(sel0, z, 0.0), axis=1, keepdims=True)
    far0 = jnp.zeros((B, 1), jnp.int32)
    jax.lax.fori_loop(0, G, body, (far0, cx0, cy0, cz0))


def _fps_centers(xyz):
    xt = jnp.transpose(xyz, (2, 0, 1))
    idx, cents = pl.pallas_call(
        _fps_body,
        in_specs=[pl.BlockSpec((3, B, N), lambda: (0, 0, 0))],
        out_specs=[
            pl.BlockSpec((B, G), lambda: (0, 0)),
            pl.BlockSpec((3, B, G), lambda: (0, 0, 0)),
        ],
        out_shape=[
            jax.ShapeDtypeStruct((B, G), jnp.int32),
            jax.ShapeDtypeStruct((3, B, G), jnp.float32),
        ],
        scratch_shapes=[pltpu.VMEM((B, N), jnp.float32)],
    )(xt)
    center = jnp.transpose(cents, (1, 2, 0))
    return idx, center


GT = 64


def _dist_body(xt_ref, c_ref, d_ref, pv_ref):
    x = xt_ref[0, 0]
    y = xt_ref[1, 0]
    z = xt_ref[2, 0]
    c = c_ref[0]
    dx = c[:, 0:1] - x
    dy = c[:, 1:2] - y
    dz = c[:, 2:3] - z
    d = (dx * dx + dy * dy) + dz * dz
    d_ref[0] = d
    cm = jnp.min(d.reshape(GT, 128, 128), axis=2)
    lane = jax.lax.broadcasted_iota(jnp.int32, (GT, 128), 1)

    def body(j, carry):
        vals, _ = carry
        m = jnp.min(vals, axis=1, keepdims=True)
        g = jnp.min(jnp.where(vals == m, lane, 128), axis=1, keepdims=True)
        vals = jnp.where(lane == g, BIG, vals)
        return vals, m

    _, t = jax.lax.fori_loop(0, K, body, (cm, jnp.zeros((GT, 1), jnp.float32)))
    msk = d <= t
    mi = jnp.where(msk, jnp.int32(1), jnp.int32(0))
    zc = jnp.zeros((GT, N), jnp.int32)
    cs = mi
    sh = 1
    while sh < N:
        cs = cs + jnp.concatenate([zc[:, :sh], cs[:, :-sh]], axis=1)
        sh *= 2
    pos = cs - mi
    pv_ref[0] = jnp.where(jnp.logical_and(msk, pos < CAP), pos, CAP)


def _distances(xt, center):
    xt = xt.reshape(3, B, 1, N)
    return pl.pallas_call(
        _dist_body,
        grid=(B, G // GT),
        in_specs=[
            pl.BlockSpec((3, 1, 1, N), lambda b, g: (0, b, 0, 0)),
            pl.BlockSpec((1, GT, 3), lambda b, g: (b, g, 0)),
        ],
        out_specs=[
            pl.BlockSpec((1, GT, N), lambda b, g: (b, g, 0)),
            pl.BlockSpec((1, GT, N), lambda b, g: (b, g, 0)),
        ],
        out_shape=[
            jax.ShapeDtypeStruct((B, G, N), jnp.float32),
            jax.ShapeDtypeStruct((B, G, N), jnp.int32),
        ],
    )(xt, center)


RW = 8
BIG = 1e30


def _extract_body(vals_ref, out_ref, sc_ref):
    W = vals_ref.shape[1]
    sc_ref[...] = vals_ref[...]
    lane = jax.lax.broadcasted_iota(jnp.int32, (RW, W), 1)
    olane = jax.lax.broadcasted_iota(jnp.int32, (RW, 128), 1)
    out_ref[...] = jnp.zeros((RW, 128), jnp.int32)

    def body(j, _):
        vals = sc_ref[...]
        m = jnp.min(vals, axis=1, keepdims=True)
        elig = vals == m
        g = jnp.min(jnp.where(elig, lane, N), axis=1, keepdims=True)
        out_ref[...] = out_ref[...] + (olane == j).astype(jnp.int32) * g
        sc_ref[...] = jnp.where(lane == g, BIG, vals)
        return 0

    jax.lax.fori_loop(0, K, body, 0)


def _topk_full(d):
    R, W = d.shape
    out = pl.pallas_call(
        _extract_body,
        grid=(R // RW,),
        in_specs=[pl.BlockSpec((RW, W), lambda i: (i, 0))],
        out_specs=pl.BlockSpec((RW, 128), lambda i: (i, 0)),
        out_shape=jax.ShapeDtypeStruct((R, 128), jnp.int32),
        scratch_shapes=[pltpu.VMEM((RW, W), jnp.float32)],
    )(d)
    return out[:, :K]


BG = B * G
NW = 32
ROWS_PW = BG // NW
CAP = 128


def _sc_compact_body(d_hbm, pv_hbm, vals_hbm, gidx_hbm,
                     buf, pbuf, candv, candi, nvec, sem):
    from jax import lax
    wid = lax.axis_index("s") * 2 + lax.axis_index("c")
    iota16 = jax.lax.iota(jnp.int32, 16)

    def row_body(r, _):
        row = wid * ROWS_PW + r
        pltpu.sync_copy(d_hbm.at[pl.ds(row * N, N)], buf)
        pltpu.sync_copy(pv_hbm.at[pl.ds(row * N, N)], pbuf)
        for j in range(8):
            candv[pl.ds(16 * j, 16)] = jnp.full((16,), BIG, jnp.float32)
            candi[pl.ds(16 * j, 16)] = jnp.zeros((16,), jnp.int32)
        nvec[...] = iota16

        def scan_step(k, _c):
            base = k * 256
            for j in range(16):
                off = base + j * 16
                v = buf[pl.ds(off, 16)]
                pv = pbuf[pl.ds(off, 16)]
                ok = pv < CAP
                plsc.store_scatter(candv, [pv], v, mask=ok)
                plsc.store_scatter(candi, [pv], nvec[...], mask=ok)
                nvec[...] = nvec[...] + 16
            return 0

        lax.fori_loop(0, 64, scan_step, 0)
        pltpu.sync_copy(candv, vals_hbm.at[pl.ds(row * CAP, CAP)])
        pltpu.sync_copy(candi, gidx_hbm.at[pl.ds(row * CAP, CAP)])
        return 0

    jax.lax.fori_loop(0, ROWS_PW, row_body, 0)


def _sc_compact(d1, pv1):
    mesh = plsc.VectorSubcoreMesh(core_axis_name="c", subcore_axis_name="s")
    f = pl.kernel(
        _sc_compact_body,
        out_type=[
            jax.ShapeDtypeStruct((BG * CAP,), jnp.float32),
            jax.ShapeDtypeStruct((BG * CAP,), jnp.int32),
        ],
        mesh=mesh,
        compiler_params=pltpu.CompilerParams(needs_layout_passes=False),
        scratch_types=[
            pltpu.VMEM((N,), jnp.float32),
            pltpu.VMEM((N,), jnp.int32),
            pltpu.VMEM((CAP,), jnp.float32),
            pltpu.VMEM((CAP,), jnp.int32),
            pltpu.VMEM((16,), jnp.int32),
            pltpu.SemaphoreType.DMA,
        ],
    )
    return f(d1, pv1)


def _extract_cand_body(vals_ref, gidx_ref, out_ref, sc_ref):
    W = vals_ref.shape[1]
    sc_ref[...] = vals_ref[...]
    gx = gidx_ref[...]
    olane = jax.lax.broadcasted_iota(jnp.int32, (RW, 128), 1)
    out_ref[...] = jnp.zeros((RW, 128), jnp.int32)

    def body(j, _):
        vals = sc_ref[...]
        m = jnp.min(vals, axis=1, keepdims=True)
        elig = vals == m
        g = jnp.min(jnp.where(elig, gx, jnp.int32(1 << 30)), axis=1, keepdims=True)
        out_ref[...] = out_ref[...] + (olane == j).astype(jnp.int32) * g
        sc_ref[...] = jnp.where(jnp.logical_and(elig, gx == g), BIG, vals)
        return 0

    jax.lax.fori_loop(0, K, body, 0)


def _topk_cand(vals, gidx):
    R, W = vals.shape
    out = pl.pallas_call(
        _extract_cand_body,
        grid=(R // RW,),
        in_specs=[
            pl.BlockSpec((RW, W), lambda i: (i, 0)),
            pl.BlockSpec((RW, W), lambda i: (i, 0)),
        ],
        out_specs=pl.BlockSpec((RW, 128), lambda i: (i, 0)),
        out_shape=jax.ShapeDtypeStruct((R, 128), jnp.int32),
        scratch_shapes=[pltpu.VMEM((RW, W), jnp.float32)],
    )(vals, gidx)
    return out[:, :K]


def _batchnorm(x, gamma, beta):
    mu = jnp.mean(x, axis=(0, 1), keepdims=True)
    var = jnp.var(x, axis=(0, 1), keepdims=True)
    return (x - mu) / jnp.sqrt(var + 1e-5) * gamma + beta


def _embedder(pg, W1, b1, g1, be1, W2, b2, W3, b3, g2, be2, W4, b4):
    bg, n, _ = pg.shape
    f = pg @ W1.T + b1
    f = jax.nn.relu(_batchnorm(f, g1, be1))
    f = f @ W2.T + b2
    fg = jnp.max(f, axis=1, keepdims=True)
    f = jnp.concatenate([jnp.broadcast_to(fg, (bg, n, fg.shape[-1])), f], axis=-1)
    f = f @ W3.T + b3
    f = jax.nn.relu(_batchnorm(f, g2, be2))
    f = f @ W4.T + b4
    return jnp.max(f, axis=1)


def _make_mask():
    base = jnp.concatenate([jnp.zeros(G - NUM_MASK), jnp.ones(NUM_MASK)])
    keys = jax.random.split(jax.random.key(42), B)
    mask = jax.vmap(lambda k: jax.random.permutation(k, base))(keys)
    return mask > 0.5


def _pos_mlp_kernel(mc_ref, pw1_ref, pb1_ref, pw2_ref, pb2_ref, out_ref):
    mc = mc_ref[0]
    hh = jnp.dot(mc, pw1_ref[...], preferred_element_type=jnp.float32) + pb1_ref[...]
    g = hh * 0.5 * (1.0 + jax.lax.erf(hh * jnp.float32(0.7071067811865476)))
    out_ref[0] = jnp.dot(g, pw2_ref[...], preferred_element_type=jnp.float32) + pb2_ref[...]


def _pos_mlp(mc, PW1, pb1, PW2, pb2):
    GP = 128
    mcp = jnp.zeros((B, GP, 8), jnp.float32).at[:, :G_VIS, :3].set(mc)
    w1 = jnp.zeros((8, 128), jnp.float32).at[:3, :].set(PW1.T)
    out = pl.pallas_call(
        _pos_mlp_kernel,
        grid=(B,),
        in_specs=[
            pl.BlockSpec((1, GP, 8), lambda i: (i, 0, 0)),
            pl.BlockSpec((8, 128), lambda i: (0, 0)),
            pl.BlockSpec((128,), lambda i: (0,)),
            pl.BlockSpec((128, C_ENC), lambda i: (0, 0)),
            pl.BlockSpec((C_ENC,), lambda i: (0,)),
        ],
        out_specs=pl.BlockSpec((1, GP, C_ENC), lambda i: (i, 0, 0)),
        out_shape=jax.ShapeDtypeStruct((B, GP, C_ENC), jnp.float32),
    )(mcp, w1, pb1, PW2.T, pb2)
    return out[:, :G_VIS, :]


def kernel(xyz, W1, b1, g1, be1, W2, b2, W3, b3, g2, be2, W4, b4, PW1, pb1, PW2, pb2):
    xt = jnp.transpose(xyz, (2, 0, 1))
    _, center = _fps_centers(xyz)
    d, pv = _distances(xt, center)
    cvals, cgidx = _sc_compact(d.reshape(BG * N), pv.reshape(BG * N))
    knn_idx = _topk_cand(cvals.reshape(BG, CAP), cgidx.reshape(BG, CAP)).reshape(B, G, K)
    neighborhood = jax.vmap(lambda pts, ind: pts[ind])(xyz, knn_idx)
    neighborhood = neighborhood - center[:, :, None, :]
    mask = _make_mask()
    vis_idx = jnp.argsort(mask.astype(jnp.int32), axis=1)[:, :G_VIS]
    tok_all = _embedder(neighborhood.reshape(B * G, K, 3), W1, b1, g1, be1, W2, b2, W3, b3, g2, be2, W4, b4).reshape(B, G, C_ENC)
    tokens = jnp.take_along_axis(tok_all, vis_idx[:, :, None], axis=1)
    mc = jnp.take_along_axis(center, vis_idx[:, :, None], axis=1)
    pos = _pos_mlp(mc, PW1, pb1, PW2, pb2)
    return tokens, pos, mask, center, neighborhood


# speedup vs baseline: 1.1588x; 1.0445x over previous
import functools

import jax
import jax.numpy as jnp
from jax.experimental import pallas as pl
from jax.experimental.pallas import tpu as pltpu
from jax.experimental.pallas import tpu_sc as plsc

B, N, G, K = 8, 16384, 256, 32
C_ENC = 384
MASK_RATIO = 0.6
NUM_MASK = int(MASK_RATIO * G)
G_VIS = G - NUM_MASK


def _fps_body(xt_ref, idx_ref, cents_ref, dists_ref):
    x = xt_ref[0]
    y = xt_ref[1]
    z = xt_ref[2]
    lane = jax.lax.broadcasted_iota(jnp.int32, (B, N), 1)
    glane = jax.lax.broadcasted_iota(jnp.int32, (B, G), 1)
    dists_ref[...] = jnp.full((B, N), 1e10, jnp.float32)
    idx_ref[...] = jnp.zeros((B, G), jnp.int32)
    cents_ref[...] = jnp.zeros((3, B, G), jnp.float32)

    def body(i, carry):
        far_i, cx, cy, cz = carry
        at_i = (glane == i).astype(jnp.int32)
        at_f = at_i.astype(jnp.float32)
        idx_ref[...] = idx_ref[...] + at_i * far_i
        cents_ref[0] = cents_ref[0] + at_f * cx
        cents_ref[1] = cents_ref[1] + at_f * cy
        cents_ref[2] = cents_ref[2] + at_f * cz
        dx = x - cx
        dy = y - cy
        dz = z - cz
        d = dx * dx + dy * dy + dz * dz
        dists = jnp.minimum(dists_ref[...], d)
        dists_ref[...] = dists
        m = jnp.max(dists, axis=1, keepdims=True)
        elig = dists == m
        nfar = jnp.min(jnp.where(elig, lane, N), axis=1, keepdims=True)
        sel = lane == nfar
        ncx = jnp.sum(jnp.where(sel, x, 0.0), axis=1, keepdims=True)
        ncy = jnp.sum(jnp.where(sel, y, 0.0), axis=1, keepdims=True)
        ncz = jnp.sum(jnp.where(sel, z, 0.0), axis=1, keepdims=True)
        return nfar, ncx, ncy, ncz

    sel0 = lane == 0
    cx0 = jnp.sum(jnp.where(sel0, x, 0.0), axis=1, keepdims=True)
    cy0 = jnp.sum(jnp.where(sel0, y, 0.0), axis=1, keepdims=True)
    cz0 = jnp.sum(jnp.where(sel0, z, 0.0), axis=1, keepdims=True)
    far0 = jnp.zeros((B, 1), jnp.int32)
    jax.lax.fori_loop(0, G, body, (far0, cx0, cy0, cz0))


def _fps_centers(xyz):
    xt = jnp.transpose(xyz, (2, 0, 1))
    idx, cents = pl.pallas_call(
        _fps_body,
        in_specs=[pl.BlockSpec((3, B, N), lambda: (0, 0, 0))],
        out_specs=[
            pl.BlockSpec((B, G), lambda: (0, 0)),
            pl.BlockSpec((3, B, G), lambda: (0, 0, 0)),
        ],
        out_shape=[
            jax.ShapeDtypeStruct((B, G), jnp.int32),
            jax.ShapeDtypeStruct((3, B, G), jnp.float32),
        ],
        scratch_shapes=[pltpu.VMEM((B, N), jnp.float32)],
    )(xt)
    center = jnp.transpose(cents, (1, 2, 0))
    return idx, center


GT = 64


def _dist_body(xt_ref, c_ref, d_ref, pv_ref):
    x = xt_ref[0, 0]
    y = xt_ref[1, 0]
    z = xt_ref[2, 0]
    c = c_ref[0]
    dx = c[:, 0:1] - x
    dy = c[:, 1:2] - y
    dz = c[:, 2:3] - z
    d = (dx * dx + dy * dy) + dz * dz
    d_ref[0] = d
    cm = jnp.min(d.reshape(GT, 128, 128), axis=2)
    lane = jax.lax.broadcasted_iota(jnp.int32, (GT, 128), 1)

    def body(j, carry):
        vals, _ = carry
        m = jnp.min(vals, axis=1, keepdims=True)
        g = jnp.min(jnp.where(vals == m, lane, 128), axis=1, keepdims=True)
        vals = jnp.where(lane == g, BIG, vals)
        return vals, m

    _, t = jax.lax.fori_loop(0, K, body, (cm, jnp.zeros((GT, 1), jnp.float32)))
    msk = d <= t
    mi = jnp.where(msk, jnp.int32(1), jnp.int32(0))
    zc = jnp.zeros((GT, N), jnp.int32)
    cs = mi
    sh = 1
    while sh < N:
        cs = cs + jnp.concatenate([zc[:, :sh], cs[:, :-sh]], axis=1)
        sh *= 2
    pos = cs - mi
    pv_ref[0] = jnp.where(jnp.logical_and(msk, pos < CAP), pos, CAP)


def _distances(xt, center):
    xt = xt.reshape(3, B, 1, N)
    return pl.pallas_call(
        _dist_body,
        grid=(B, G // GT),
        in_specs=[
            pl.BlockSpec((3, 1, 1, N), lambda b, g: (0, b, 0, 0)),
            pl.BlockSpec((1, GT, 3), lambda b, g: (b, g, 0)),
        ],
        out_specs=[
            pl.BlockSpec((1, GT, N), lambda b, g: (b, g, 0)),
            pl.BlockSpec((1, GT, N), lambda b, g: (b, g, 0)),
        ],
        out_shape=[
            jax.ShapeDtypeStruct((B, G, N), jnp.float32),
            jax.ShapeDtypeStruct((B, G, N), jnp.int32),
        ],
    )(xt, center)


RW = 8
BIG = 1e30


def _extract_body(vals_ref, out_ref, sc_ref):
    W = vals_ref.shape[1]
    sc_ref[...] = vals_ref[...]
    lane = jax.lax.broadcasted_iota(jnp.int32, (RW, W), 1)
    olane = jax.lax.broadcasted_iota(jnp.int32, (RW, 128), 1)
    out_ref[...] = jnp.zeros((RW, 128), jnp.int32)

    def body(j, _):
        vals = sc_ref[...]
        m = jnp.min(vals, axis=1, keepdims=True)
        elig = vals == m
        g = jnp.min(jnp.where(elig, lane, N), axis=1, keepdims=True)
        out_ref[...] = out_ref[...] + (olane == j).astype(jnp.int32) * g
        sc_ref[...] = jnp.where(lane == g, BIG, vals)
        return 0

    jax.lax.fori_loop(0, K, body, 0)


def _topk_full(d):
    R, W = d.shape
    out = pl.pallas_call(
        _extract_body,
        grid=(R // RW,),
        in_specs=[pl.BlockSpec((RW, W), lambda i: (i, 0))],
        out_specs=pl.BlockSpec((RW, 128), lambda i: (i, 0)),
        out_shape=jax.ShapeDtypeStruct((R, 128), jnp.int32),
        scratch_shapes=[pltpu.VMEM((RW, W), jnp.float32)],
    )(d)
    return out[:, :K]


BG = B * G
NW = 32
ROWS_PW = BG // NW
CAP = 128


def _sc_compact_body(d_hbm, pv_hbm, vals_hbm, gidx_hbm,
                     buf, pbuf, candv, candi, nvec, sem):
    from jax import lax
    wid = lax.axis_index("s") * 2 + lax.axis_index("c")
    iota16 = jax.lax.iota(jnp.int32, 16)

    def row_body(r, _):
        row = wid * ROWS_PW + r
        pltpu.sync_copy(d_hbm.at[row], buf)
        pltpu.sync_copy(pv_hbm.at[row], pbuf)
        for j in range(8):
            candv[pl.ds(16 * j, 16)] = jnp.full((16,), BIG, jnp.float32)
            candi[pl.ds(16 * j, 16)] = jnp.zeros((16,), jnp.int32)
        nvec[...] = iota16

        def scan_step(k, _c):
            base = k * 256
            for j in range(16):
                off = base + j * 16
                v = buf[pl.ds(off, 16)]
                pv = pbuf[pl.ds(off, 16)]
                ok = pv < CAP
                plsc.store_scatter(candv, [pv], v, mask=ok)
                plsc.store_scatter(candi, [pv], nvec[...], mask=ok)
                nvec[...] = nvec[...] + 16
            return 0

        lax.fori_loop(0, 64, scan_step, 0)
        pltpu.sync_copy(candv, vals_hbm.at[row])
        pltpu.sync_copy(candi, gidx_hbm.at[row])
        return 0

    jax.lax.fori_loop(0, ROWS_PW, row_body, 0)


def _sc_compact(d1, pv1):
    mesh = plsc.VectorSubcoreMesh(core_axis_name="c", subcore_axis_name="s")
    f = pl.kernel(
        _sc_compact_body,
        out_type=[
            jax.ShapeDtypeStruct((BG, CAP), jnp.float32),
            jax.ShapeDtypeStruct((BG, CAP), jnp.int32),
        ],
        mesh=mesh,
        compiler_params=pltpu.CompilerParams(needs_layout_passes=False),
        scratch_types=[
            pltpu.VMEM((N,), jnp.float32),
            pltpu.VMEM((N,), jnp.int32),
            pltpu.VMEM((CAP,), jnp.float32),
            pltpu.VMEM((CAP,), jnp.int32),
            pltpu.VMEM((16,), jnp.int32),
            pltpu.SemaphoreType.DMA,
        ],
    )
    return f(d1, pv1)


def _extract_cand_body(vals_ref, gidx_ref, out_ref, sc_ref):
    W = vals_ref.shape[1]
    sc_ref[...] = vals_ref[...]
    gx = gidx_ref[...]
    olane = jax.lax.broadcasted_iota(jnp.int32, (RW, 128), 1)
    out_ref[...] = jnp.zeros((RW, 128), jnp.int32)

    def body(j, _):
        vals = sc_ref[...]
        m = jnp.min(vals, axis=1, keepdims=True)
        elig = vals == m
        g = jnp.min(jnp.where(elig, gx, jnp.int32(1 << 30)), axis=1, keepdims=True)
        out_ref[...] = out_ref[...] + (olane == j).astype(jnp.int32) * g
        sc_ref[...] = jnp.where(jnp.logical_and(elig, gx == g), BIG, vals)
        return 0

    jax.lax.fori_loop(0, K, body, 0)


def _topk_cand(vals, gidx):
    R, W = vals.shape
    out = pl.pallas_call(
        _extract_cand_body,
        grid=(R // RW,),
        in_specs=[
            pl.BlockSpec((RW, W), lambda i: (i, 0)),
            pl.BlockSpec((RW, W), lambda i: (i, 0)),
        ],
        out_specs=pl.BlockSpec((RW, 128), lambda i: (i, 0)),
        out_shape=jax.ShapeDtypeStruct((R, 128), jnp.int32),
        scratch_shapes=[pltpu.VMEM((RW, W), jnp.float32)],
    )(vals, gidx)
    return out[:, :K]


def _batchnorm(x, gamma, beta):
    mu = jnp.mean(x, axis=(0, 1), keepdims=True)
    var = jnp.var(x, axis=(0, 1), keepdims=True)
    return (x - mu) / jnp.sqrt(var + 1e-5) * gamma + beta


def _embedder(pg, W1, b1, g1, be1, W2, b2, W3, b3, g2, be2, W4, b4):
    bg, n, _ = pg.shape
    f = pg @ W1.T + b1
    f = jax.nn.relu(_batchnorm(f, g1, be1))
    f = f @ W2.T + b2
    fg = jnp.max(f, axis=1, keepdims=True)
    f = jnp.concatenate([jnp.broadcast_to(fg, (bg, n, fg.shape[-1])), f], axis=-1)
    f = f @ W3.T + b3
    f = jax.nn.relu(_batchnorm(f, g2, be2))
    f = f @ W4.T + b4
    return jnp.max(f, axis=1)


def _make_mask():
    base = jnp.concatenate([jnp.zeros(G - NUM_MASK), jnp.ones(NUM_MASK)])
    keys = jax.random.split(jax.random.key(42), B)
    mask = jax.vmap(lambda k: jax.random.permutation(k, base))(keys)
    return mask > 0.5


def _pos_mlp_kernel(mc_ref, pw1_ref, pb1_ref, pw2_ref, pb2_ref, out_ref):
    mc = mc_ref[0]
    hh = jnp.dot(mc, pw1_ref[...], preferred_element_type=jnp.float32) + pb1_ref[...]
    g = hh * 0.5 * (1.0 + jax.lax.erf(hh * jnp.float32(0.7071067811865476)))
    out_ref[0] = jnp.dot(g, pw2_ref[...], preferred_element_type=jnp.float32) + pb2_ref[...]


def _pos_mlp(mc, PW1, pb1, PW2, pb2):
    GP = 128
    mcp = jnp.zeros((B, GP, 8), jnp.float32).at[:, :G_VIS, :3].set(mc)
    w1 = jnp.zeros((8, 128), jnp.float32).at[:3, :].set(PW1.T)
    out = pl.pallas_call(
        _pos_mlp_kernel,
        grid=(B,),
        in_specs=[
            pl.BlockSpec((1, GP, 8), lambda i: (i, 0, 0)),
            pl.BlockSpec((8, 128), lambda i: (0, 0)),
            pl.BlockSpec((128,), lambda i: (0,)),
            pl.BlockSpec((128, C_ENC), lambda i: (0, 0)),
            pl.BlockSpec((C_ENC,), lambda i: (0,)),
        ],
        out_specs=pl.BlockSpec((1, GP, C_ENC), lambda i: (i, 0, 0)),
        out_shape=jax.ShapeDtypeStruct((B, GP, C_ENC), jnp.float32),
    )(mcp, w1, pb1, PW2.T, pb2)
    return out[:, :G_VIS, :]


def kernel(xyz, W1, b1, g1, be1, W2, b2, W3, b3, g2, be2, W4, b4, PW1, pb1, PW2, pb2):
    xt = jnp.transpose(xyz, (2, 0, 1))
    _, center = _fps_centers(xyz)
    d, pv = _distances(xt, center)
    cvals, cgidx = _sc_compact(d.reshape(BG, N), pv.reshape(BG, N))
    knn_idx = _topk_cand(cvals, cgidx).reshape(B, G, K)
    neighborhood = jax.vmap(lambda pts, ind: pts[ind])(xyz, knn_idx)
    neighborhood = neighborhood - center[:, :, None, :]
    mask = _make_mask()
    vis_idx = jnp.argsort(mask.astype(jnp.int32), axis=1)[:, :G_VIS]
    tok_all = _embedder(neighborhood.reshape(B * G, K, 3), W1, b1, g1, be1, W2, b2, W3, b3, g2, be2, W4, b4).reshape(B, G, C_ENC)
    tokens = jnp.take_along_axis(tok_all, vis_idx[:, :, None], axis=1)
    mc = jnp.take_along_axis(center, vis_idx[:, :, None], axis=1)
    pos = _pos_mlp(mc, PW1, pb1, PW2, pb2)
    return tokens, pos, mask, center, neighborhood
